# Initial kernel scaffold; baseline (speedup 1.0000x reference)
import jax
import jax.numpy as jnp
from jax.experimental import pallas as pl


def kernel(edge_index, edge_weight, features, W1, b1, W2, b2):
    raise NotImplementedError("write your pallas kernel here")


# speedup vs baseline: 4.0266x; 4.0266x over previous
import dataclasses
import functools

import jax
import jax.numpy as jnp
from jax import lax
from jax.experimental import pallas as pl
from jax.experimental.pallas import tpu as pltpu
from jax.experimental.pallas import tpu_sc as plsc

N_NODES = 10000
N_EDGES = 320000
D = 128

NC = 2
NS = 16
LANES = 16
CHUNK = 80
EDGES_PER_TILE = N_EDGES // (NC * NS)
CHUNKS_PER_TILE = EDGES_PER_TILE // CHUNK
ROWS_MAIN = 624
ROWS_LAST = N_NODES - (NS - 1) * ROWS_MAIN


def _sc_spmm(dst, src, wgt, features, zeros):
    mesh = plsc.VectorSubcoreMesh(
        core_axis_name="c", subcore_axis_name="s", num_cores=NC, num_subcores=NS
    )
    cp = pltpu.CompilerParams()
    if "needs_layout_passes" in pltpu.CompilerParams.__dataclass_fields__:
        cp = dataclasses.replace(cp, needs_layout_passes=False)

    @functools.partial(
        pl.kernel,
        compiler_params=cp,
        out_type=jax.ShapeDtypeStruct((NC * N_NODES, D), jnp.float32),
        mesh=mesh,
        scratch_types=[
            pltpu.VMEM((CHUNK,), jnp.int32),
            pltpu.VMEM((CHUNK,), jnp.int32),
            pltpu.VMEM((CHUNK,), jnp.float32),
            pltpu.VMEM((CHUNK, D), jnp.float32),
            pltpu.VMEM_SHARED((N_NODES, D), jnp.float32),
        ],
    )
    def spmm(dst_hbm, src_hbm, w_hbm, f_hbm, z_hbm, out_hbm,
             src_v, dst_v, w_v, rows_v, acc_sh):
        c = lax.axis_index("c")
        s = lax.axis_index("s")

        row_off = s * ROWS_MAIN

        @pl.when(s < NS - 1)
        def _():
            pltpu.sync_copy(z_hbm.at[pl.ds(row_off, ROWS_MAIN)],
                            acc_sh.at[pl.ds(row_off, ROWS_MAIN)])

        @pl.when(s == NS - 1)
        def _():
            pltpu.sync_copy(z_hbm.at[pl.ds(row_off, ROWS_LAST)],
                            acc_sh.at[pl.ds(row_off, ROWS_LAST)])

        plsc.subcore_barrier()

        tile_base = c * (NS * EDGES_PER_TILE) + s * EDGES_PER_TILE

        @pl.loop(0, CHUNKS_PER_TILE)
        def _(i):
            base = tile_base + i * CHUNK
            pltpu.sync_copy(src_hbm.at[pl.ds(base, CHUNK)], src_v)
            pltpu.sync_copy(dst_hbm.at[pl.ds(base, CHUNK)], dst_v)
            pltpu.sync_copy(w_hbm.at[pl.ds(base, CHUNK)], w_v)
            pltpu.sync_copy(f_hbm.at[src_v], rows_v)

            @pl.loop(0, CHUNK)
            def _(e):
                widx = jnp.full((LANES,), e, dtype=jnp.int32)
                wv = plsc.load_gather(w_v, [widx])
                for g in range(D // LANES):
                    sl = (e, pl.ds(g * LANES, LANES))
                    rows_v[sl] = rows_v[sl] * wv

            pltpu.sync_copy(rows_v, acc_sh.at[dst_v], add=True)

        plsc.subcore_barrier()
        out_base = c * N_NODES + row_off

        @pl.when(s < NS - 1)
        def _():
            pltpu.sync_copy(acc_sh.at[pl.ds(row_off, ROWS_MAIN)],
                            out_hbm.at[pl.ds(out_base, ROWS_MAIN)])

        @pl.when(s == NS - 1)
        def _():
            pltpu.sync_copy(acc_sh.at[pl.ds(row_off, ROWS_LAST)],
                            out_hbm.at[pl.ds(out_base, ROWS_LAST)])

    return spmm(dst, src, wgt, features, zeros)


def _tc_combine_body(parts_ref, f_ref, w1_ref, w2_ref, bias_ref, out_ref):
    x = parts_ref[0] + parts_ref[1]
    f = f_ref[...]
    out_ref[...] = (
        jnp.dot(f + x, w1_ref[...], preferred_element_type=jnp.float32)
        + jnp.dot(x * f, w2_ref[...], preferred_element_type=jnp.float32)
        + bias_ref[...]
    )


def _tc_combine(parts, features, W1, W2, bias):
    blk = 1000
    grid = (N_NODES // blk,)
    return pl.pallas_call(
        _tc_combine_body,
        grid=grid,
        in_specs=[
            pl.BlockSpec((2, blk, D), lambda i: (0, i, 0)),
            pl.BlockSpec((blk, D), lambda i: (i, 0)),
            pl.BlockSpec((D, D), lambda i: (0, 0)),
            pl.BlockSpec((D, D), lambda i: (0, 0)),
            pl.BlockSpec((1, D), lambda i: (0, 0)),
        ],
        out_specs=pl.BlockSpec((blk, D), lambda i: (i, 0)),
        out_shape=jax.ShapeDtypeStruct((N_NODES, D), jnp.float32),
    )(parts, features, W1, W2, bias)


@jax.jit
def kernel(edge_index, edge_weight, features, W1, b1, W2, b2):
    dst = edge_index[0]
    src = edge_index[1]
    zeros = jnp.zeros((N_NODES, D), jnp.float32)
    parts = _sc_spmm(dst, src, edge_weight, features, zeros)
    parts = parts.reshape(NC, N_NODES, D)
    bias = (b1 + b2).reshape(1, D)
    return _tc_combine(parts, features, W1, W2, bias)


# speedup vs baseline: 4.6013x; 1.1427x over previous
import dataclasses
import functools

import jax
import jax.numpy as jnp
from jax import lax
from jax.experimental import pallas as pl
from jax.experimental.pallas import tpu as pltpu
from jax.experimental.pallas import tpu_sc as plsc

N_NODES = 10000
N_EDGES = 320000
D = 128

NC = 2
NS = 16
NW = NC * NS
LANES = 16
CHUNK = 80
EDGES_PER_TILE = N_EDGES // NW
NCHUNK = EDGES_PER_TILE // CHUNK
ROWS_MAIN = 624
ROWS_LAST = N_NODES - (NS - 1) * ROWS_MAIN


def _sc_spmm(dst, src, wgt, features, zeros):
    mesh = plsc.VectorSubcoreMesh(
        core_axis_name="c", subcore_axis_name="s", num_cores=NC, num_subcores=NS
    )
    cp = pltpu.CompilerParams()
    if "needs_layout_passes" in pltpu.CompilerParams.__dataclass_fields__:
        cp = dataclasses.replace(cp, needs_layout_passes=False)

    @functools.partial(
        pl.kernel,
        compiler_params=cp,
        out_type=jax.ShapeDtypeStruct((NC * N_NODES, D), jnp.float32),
        mesh=mesh,
        scratch_types=[
            [pltpu.VMEM((CHUNK,), jnp.int32) for _ in range(4)],
            [pltpu.VMEM((CHUNK,), jnp.int32) for _ in range(4)],
            [pltpu.VMEM((CHUNK,), jnp.float32) for _ in range(4)],
            [pltpu.VMEM((CHUNK, D), jnp.float32) for _ in range(2)],
            [pltpu.VMEM((CHUNK, D), jnp.float32) for _ in range(2)],
            pltpu.VMEM_SHARED((N_NODES, D), jnp.float32),
            [pltpu.SemaphoreType.DMA for _ in range(4)],
            [pltpu.SemaphoreType.DMA for _ in range(2)],
            [pltpu.SemaphoreType.DMA for _ in range(2)],
        ],
    )
    def spmm(dst_hbm, src_hbm, w_hbm, f_hbm, z_hbm, out_hbm,
             src_v, dst_v, w_v, rg, rs, acc_sh, isem, gsem, ssem):
        c = lax.axis_index("c")
        s = lax.axis_index("s")
        t = c * NS + s
        tile_base = t * EDGES_PER_TILE

        row_off = s * ROWS_MAIN

        @pl.when(s < NS - 1)
        def _():
            pltpu.sync_copy(z_hbm.at[pl.ds(row_off, ROWS_MAIN)],
                            acc_sh.at[pl.ds(row_off, ROWS_MAIN)])

        @pl.when(s == NS - 1)
        def _():
            pltpu.sync_copy(z_hbm.at[pl.ds(row_off, ROWS_LAST)],
                            acc_sh.at[pl.ds(row_off, ROWS_LAST)])

        plsc.subcore_barrier()

        def issue_idx(i, q):
            base = tile_base + i * CHUNK
            pltpu.async_copy(src_hbm.at[pl.ds(base, CHUNK)], src_v[q], isem[q])
            pltpu.async_copy(dst_hbm.at[pl.ds(base, CHUNK)], dst_v[q], isem[q])
            pltpu.async_copy(w_hbm.at[pl.ds(base, CHUNK)], w_v[q], isem[q])

        def wait_idx(q):
            pltpu.make_async_copy(src_hbm.at[pl.ds(0, CHUNK)], src_v[q],
                                  isem[q]).wait()
            pltpu.make_async_copy(dst_hbm.at[pl.ds(0, CHUNK)], dst_v[q],
                                  isem[q]).wait()
            pltpu.make_async_copy(w_hbm.at[pl.ds(0, CHUNK)], w_v[q],
                                  isem[q]).wait()

        def issue_gather(q, p):
            pltpu.async_copy(f_hbm.at[src_v[q]], rg[p], gsem[p])

        def wait_rows(buf, sem):
            pltpu.make_async_copy(f_hbm.at[src_v[0]], buf, sem).wait()

        def scale(q, p):
            @pl.loop(0, CHUNK)
            def _(e):
                wv = plsc.load_gather(w_v[q], [jnp.full((LANES,), e, jnp.int32)])
                for g in range(D // LANES):
                    sl = (e, pl.ds(g * LANES, LANES))
                    rs[p][sl] = rg[p][sl] * wv

        def issue_scatter(q, p):
            pltpu.async_copy(rs[p], acc_sh.at[dst_v[q]], ssem[p], add=True)

        issue_idx(0, 0)
        issue_idx(1, 1)
        wait_idx(0)
        wait_idx(1)
        issue_gather(0, 0)
        issue_gather(1, 1)

        def slot(i, q, p, qn):
            issue_idx(i + 2, qn)
            wait_rows(rg[p], gsem[p])
            scale(q, p)
            issue_scatter(q, p)
            wait_idx(qn)
            issue_gather(qn, p)

        slot(0, 0, 0, 2)
        slot(1, 1, 1, 3)

        @pl.loop(0, (NCHUNK - 5) // 4)
        def _(j):
            i = 4 * j + 2
            for k in range(4):
                q = (2 + k) % 4
                p = k % 2
                wait_rows(rs[p], ssem[p])
                slot(i + k, q, p, k)

        wait_rows(rs[0], ssem[0])
        slot(NCHUNK - 3, 2, 0, 0)
        wait_rows(rs[1], ssem[1])
        wait_rows(rg[1], gsem[1])
        scale(3, 1)
        issue_scatter(3, 1)
        wait_rows(rs[0], ssem[0])
        wait_rows(rg[0], gsem[0])
        scale(0, 0)
        issue_scatter(0, 0)

        wait_rows(rs[1], ssem[1])
        wait_rows(rs[0], ssem[0])
        plsc.subcore_barrier()

        out_base = c * N_NODES + row_off

        @pl.when(s < NS - 1)
        def _():
            pltpu.sync_copy(acc_sh.at[pl.ds(row_off, ROWS_MAIN)],
                            out_hbm.at[pl.ds(out_base, ROWS_MAIN)])

        @pl.when(s == NS - 1)
        def _():
            pltpu.sync_copy(acc_sh.at[pl.ds(row_off, ROWS_LAST)],
                            out_hbm.at[pl.ds(out_base, ROWS_LAST)])

    return spmm(dst, src, wgt, features, zeros)


def _tc_combine_body(parts_ref, f_ref, w1_ref, w2_ref, bias_ref, out_ref):
    x = parts_ref[0] + parts_ref[1]
    f = f_ref[...]
    out_ref[...] = (
        jnp.dot(f + x, w1_ref[...], preferred_element_type=jnp.float32)
        + jnp.dot(x * f, w2_ref[...], preferred_element_type=jnp.float32)
        + bias_ref[...]
    )


def _tc_combine(parts, features, W1, W2, bias):
    blk = 1000
    grid = (N_NODES // blk,)
    return pl.pallas_call(
        _tc_combine_body,
        grid=grid,
        in_specs=[
            pl.BlockSpec((2, blk, D), lambda i: (0, i, 0)),
            pl.BlockSpec((blk, D), lambda i: (i, 0)),
            pl.BlockSpec((D, D), lambda i: (0, 0)),
            pl.BlockSpec((D, D), lambda i: (0, 0)),
            pl.BlockSpec((1, D), lambda i: (0, 0)),
        ],
        out_specs=pl.BlockSpec((blk, D), lambda i: (i, 0)),
        out_shape=jax.ShapeDtypeStruct((N_NODES, D), jnp.float32),
    )(parts, features, W1, W2, bias)


@jax.jit
def kernel(edge_index, edge_weight, features, W1, b1, W2, b2):
    dst = edge_index[0]
    src = edge_index[1]
    zeros = jnp.zeros((N_NODES, D), jnp.float32)
    parts = _sc_spmm(dst, src, edge_weight, features, zeros)
    parts = parts.reshape(NC, N_NODES, D)
    bias = (b1 + b2).reshape(1, D)
    return _tc_combine(parts, features, W1, W2, bias)


# speedup vs baseline: 6.1126x; 1.3284x over previous
import dataclasses
import functools

import jax
import jax.numpy as jnp
import numpy as np
from jax import lax
from jax.experimental import pallas as pl
from jax.experimental.pallas import tpu as pltpu
from jax.experimental.pallas import tpu_sc as plsc

N_NODES = 10000
N_EDGES = 320000
D = 128

_PERM = np.empty((D,), dtype=np.int32)
for _g in range(D // 32):
    for _k in range(16):
        _PERM[32 * _g + 2 * _k] = 32 * _g + _k
        _PERM[32 * _g + 2 * _k + 1] = 32 * _g + 16 + _k

NC = 2
NS = 16
NW = NC * NS
LANES = 16
CHUNK = 80
EDGES_PER_TILE = N_EDGES // NW
NCHUNK = EDGES_PER_TILE // CHUNK
ROWS_MAIN = 624
ROWS_LAST = N_NODES - (NS - 1) * ROWS_MAIN


def _sc_spmm(dst, src, wgt, features, zeros):
    mesh = plsc.VectorSubcoreMesh(
        core_axis_name="c", subcore_axis_name="s", num_cores=NC, num_subcores=NS
    )
    cp = pltpu.CompilerParams()
    if "needs_layout_passes" in pltpu.CompilerParams.__dataclass_fields__:
        cp = dataclasses.replace(cp, needs_layout_passes=False)
    if "use_tc_tiling_on_sc" in pltpu.CompilerParams.__dataclass_fields__:
        cp = dataclasses.replace(cp, use_tc_tiling_on_sc=False)

    @functools.partial(
        pl.kernel,
        compiler_params=cp,
        out_type=jax.ShapeDtypeStruct((NC * N_NODES, D), jnp.float32),
        mesh=mesh,
        scratch_types=[
            [pltpu.VMEM((CHUNK,), jnp.int32) for _ in range(4)],
            [pltpu.VMEM((CHUNK,), jnp.int32) for _ in range(4)],
            [pltpu.VMEM((CHUNK,), jnp.float32) for _ in range(4)],
            [pltpu.VMEM((CHUNK, D // 2), jnp.int32) for _ in range(2)],
            [pltpu.VMEM((CHUNK, D), jnp.float32) for _ in range(2)],
            pltpu.VMEM_SHARED((N_NODES, D), jnp.float32),
            [pltpu.SemaphoreType.DMA for _ in range(4)],
            [pltpu.SemaphoreType.DMA for _ in range(2)],
            [pltpu.SemaphoreType.DMA for _ in range(2)],
        ],
    )
    def spmm(dst_hbm, src_hbm, w_hbm, f_hbm, z_hbm, out_hbm,
             src_v, dst_v, w_v, rg, rs, acc_sh, isem, gsem, ssem):
        c = lax.axis_index("c")
        s = lax.axis_index("s")
        t = c * NS + s
        tile_base = t * EDGES_PER_TILE

        row_off = s * ROWS_MAIN

        @pl.when(s < NS - 1)
        def _():
            pltpu.sync_copy(z_hbm.at[pl.ds(row_off, ROWS_MAIN)],
                            acc_sh.at[pl.ds(row_off, ROWS_MAIN)])

        @pl.when(s == NS - 1)
        def _():
            pltpu.sync_copy(z_hbm.at[pl.ds(row_off, ROWS_LAST)],
                            acc_sh.at[pl.ds(row_off, ROWS_LAST)])

        plsc.subcore_barrier()

        def issue_idx(i, q):
            base = tile_base + i * CHUNK
            pltpu.async_copy(src_hbm.at[pl.ds(base, CHUNK)], src_v[q], isem[q])
            pltpu.async_copy(dst_hbm.at[pl.ds(base, CHUNK)], dst_v[q], isem[q])
            pltpu.async_copy(w_hbm.at[pl.ds(base, CHUNK)], w_v[q], isem[q])

        def wait_idx(q):
            pltpu.make_async_copy(src_hbm.at[pl.ds(0, CHUNK)], src_v[q],
                                  isem[q]).wait()
            pltpu.make_async_copy(dst_hbm.at[pl.ds(0, CHUNK)], dst_v[q],
                                  isem[q]).wait()
            pltpu.make_async_copy(w_hbm.at[pl.ds(0, CHUNK)], w_v[q],
                                  isem[q]).wait()

        def issue_gather(q, p):
            pltpu.async_copy(f_hbm.at[src_v[q]], rg[p], gsem[p])

        def wait_gather(p):
            pltpu.make_async_copy(f_hbm.at[src_v[0]], rg[p], gsem[p]).wait()

        def wait_scatter(p):
            pltpu.make_async_copy(rs[p], acc_sh.at[dst_v[0]], ssem[p]).wait()

        def scale(q, p):
            @pl.loop(0, CHUNK)
            def _(e):
                wv = plsc.load_gather(w_v[q], [jnp.full((LANES,), e, jnp.int32)])
                for g in range(D // (2 * LANES)):
                    xi = rg[p][e, pl.ds(g * LANES, LANES)]
                    lo = plsc.bitcast(xi << 16, jnp.float32)
                    hi = plsc.bitcast(xi & jnp.int32(-65536), jnp.float32)
                    rs[p][e, pl.ds(g * 2 * LANES, LANES)] = lo * wv
                    rs[p][e, pl.ds(g * 2 * LANES + LANES, LANES)] = hi * wv

        def issue_scatter(q, p):
            pltpu.async_copy(rs[p], acc_sh.at[dst_v[q]], ssem[p], add=True)

        issue_idx(0, 0)
        issue_idx(1, 1)
        wait_idx(0)
        wait_idx(1)
        issue_gather(0, 0)
        issue_gather(1, 1)

        def slot(i, q, p, qn):
            issue_idx(i + 2, qn)
            wait_gather(p)
            scale(q, p)
            issue_scatter(q, p)
            wait_idx(qn)
            issue_gather(qn, p)

        slot(0, 0, 0, 2)
        slot(1, 1, 1, 3)

        @pl.loop(0, (NCHUNK - 5) // 4)
        def _(j):
            i = 4 * j + 2
            for k in range(4):
                q = (2 + k) % 4
                p = k % 2
                wait_scatter(p)
                slot(i + k, q, p, k)

        wait_scatter(0)
        slot(NCHUNK - 3, 2, 0, 0)
        wait_scatter(1)
        wait_gather(1)
        scale(3, 1)
        issue_scatter(3, 1)
        wait_scatter(0)
        wait_gather(0)
        scale(0, 0)
        issue_scatter(0, 0)

        wait_scatter(1)
        wait_scatter(0)
        plsc.subcore_barrier()

        out_base = c * N_NODES + row_off

        @pl.when(s < NS - 1)
        def _():
            pltpu.sync_copy(acc_sh.at[pl.ds(row_off, ROWS_MAIN)],
                            out_hbm.at[pl.ds(out_base, ROWS_MAIN)])

        @pl.when(s == NS - 1)
        def _():
            pltpu.sync_copy(acc_sh.at[pl.ds(row_off, ROWS_LAST)],
                            out_hbm.at[pl.ds(out_base, ROWS_LAST)])

    return spmm(dst, src, wgt, features, zeros)


def _tc_combine_body(parts_ref, f_ref, w1_ref, w2_ref, bias_ref, out_ref):
    x = parts_ref[0] + parts_ref[1]
    f = f_ref[...]
    out_ref[...] = (
        jnp.dot(f + x, w1_ref[...], preferred_element_type=jnp.float32)
        + jnp.dot(x * f, w2_ref[...], preferred_element_type=jnp.float32)
        + bias_ref[...]
    )


def _tc_combine(parts, features, W1, W2, bias):
    blk = 1000
    grid = (N_NODES // blk,)
    return pl.pallas_call(
        _tc_combine_body,
        grid=grid,
        in_specs=[
            pl.BlockSpec((2, blk, D), lambda i: (0, i, 0)),
            pl.BlockSpec((blk, D), lambda i: (i, 0)),
            pl.BlockSpec((D, D), lambda i: (0, 0)),
            pl.BlockSpec((D, D), lambda i: (0, 0)),
            pl.BlockSpec((1, D), lambda i: (0, 0)),
        ],
        out_specs=pl.BlockSpec((blk, D), lambda i: (i, 0)),
        out_shape=jax.ShapeDtypeStruct((N_NODES, D), jnp.float32),
    )(parts, features, W1, W2, bias)


@jax.jit
def kernel(edge_index, edge_weight, features, W1, b1, W2, b2):
    dst = edge_index[0]
    src = edge_index[1]
    f_bf = features.astype(jnp.bfloat16)[:, _PERM]
    f_i32 = lax.bitcast_convert_type(
        f_bf.reshape(N_NODES, D // 2, 2), jnp.int32)
    zeros = jnp.zeros((N_NODES, D), jnp.float32)
    parts = _sc_spmm(dst, src, edge_weight, f_i32, zeros)
    parts = parts.reshape(NC, N_NODES, D)
    bias = (b1 + b2).reshape(1, D)
    return _tc_combine(parts, features, W1, W2, bias)


# speedup vs baseline: 11.0089x; 1.8010x over previous
import dataclasses
import functools

import jax
import jax.numpy as jnp
import numpy as np
from jax import lax
from jax.experimental import pallas as pl
from jax.experimental.pallas import tpu as pltpu
from jax.experimental.pallas import tpu_sc as plsc

N_NODES = 10000
N_EDGES = 320000
D = 128

_PERM = np.empty((D,), dtype=np.int32)
for _g in range(D // 32):
    for _k in range(16):
        _PERM[32 * _g + 2 * _k] = 32 * _g + _k
        _PERM[32 * _g + 2 * _k + 1] = 32 * _g + 16 + _k

NC = 2
NS = 16
NW = NC * NS
LANES = 16
CHUNK = 80
EDGES_PER_TILE = N_EDGES // NW
NCHUNK = EDGES_PER_TILE // CHUNK
ROWS_MAIN = 624
ROWS_LAST = N_NODES - (NS - 1) * ROWS_MAIN


def _sc_spmm(dst, src, wgt, features, zeros):
    mesh = plsc.VectorSubcoreMesh(
        core_axis_name="c", subcore_axis_name="s", num_cores=NC, num_subcores=NS
    )
    cp = pltpu.CompilerParams()
    if "needs_layout_passes" in pltpu.CompilerParams.__dataclass_fields__:
        cp = dataclasses.replace(cp, needs_layout_passes=False)
    if "use_tc_tiling_on_sc" in pltpu.CompilerParams.__dataclass_fields__:
        cp = dataclasses.replace(cp, use_tc_tiling_on_sc=False)

    @functools.partial(
        pl.kernel,
        compiler_params=cp,
        out_type=jax.ShapeDtypeStruct((NC * N_NODES, D), jnp.float32),
        mesh=mesh,
        scratch_types=[
            [pltpu.VMEM((CHUNK,), jnp.int32) for _ in range(4)],
            [pltpu.VMEM((CHUNK,), jnp.int32) for _ in range(4)],
            [pltpu.VMEM((CHUNK,), jnp.float32) for _ in range(4)],
            [pltpu.VMEM((CHUNK, D // 2), jnp.int32) for _ in range(2)],
            [pltpu.VMEM((CHUNK, D), jnp.float32) for _ in range(2)],
            pltpu.VMEM_SHARED((N_NODES, D), jnp.float32),
            [pltpu.SemaphoreType.DMA for _ in range(4)],
            [pltpu.SemaphoreType.DMA for _ in range(2)],
            [pltpu.SemaphoreType.DMA for _ in range(2)],
        ],
    )
    def spmm(dst_hbm, src_hbm, w_hbm, f_hbm, z_hbm, out_hbm,
             src_v, dst_v, w_v, rg, rs, acc_sh, isem, gsem, ssem):
        c = lax.axis_index("c")
        s = lax.axis_index("s")
        t = c * NS + s
        tile_base = t * EDGES_PER_TILE

        row_off = s * ROWS_MAIN

        @pl.when(s < NS - 1)
        def _():
            pltpu.sync_copy(z_hbm.at[pl.ds(row_off, ROWS_MAIN)],
                            acc_sh.at[pl.ds(row_off, ROWS_MAIN)])

        @pl.when(s == NS - 1)
        def _():
            pltpu.sync_copy(z_hbm.at[pl.ds(row_off, ROWS_LAST)],
                            acc_sh.at[pl.ds(row_off, ROWS_LAST)])

        plsc.subcore_barrier()

        def issue_idx(i, q):
            base = tile_base + i * CHUNK
            pltpu.async_copy(src_hbm.at[pl.ds(base, CHUNK)], src_v[q], isem[q])
            pltpu.async_copy(dst_hbm.at[pl.ds(base, CHUNK)], dst_v[q], isem[q])
            pltpu.async_copy(w_hbm.at[pl.ds(base, CHUNK)], w_v[q], isem[q])

        def wait_idx(q):
            pltpu.make_async_copy(src_hbm.at[pl.ds(0, CHUNK)], src_v[q],
                                  isem[q]).wait()
            pltpu.make_async_copy(dst_hbm.at[pl.ds(0, CHUNK)], dst_v[q],
                                  isem[q]).wait()
            pltpu.make_async_copy(w_hbm.at[pl.ds(0, CHUNK)], w_v[q],
                                  isem[q]).wait()

        def issue_gather(q, p):
            pltpu.async_copy(f_hbm.at[src_v[q]], rg[p], gsem[p])

        def wait_gather(p):
            pltpu.make_async_copy(f_hbm.at[src_v[0]], rg[p], gsem[p]).wait()

        def wait_scatter(p):
            pltpu.make_async_copy(rs[p], acc_sh.at[dst_v[0]], ssem[p]).wait()

        def scale(q, p):
            @plsc.parallel_loop(0, CHUNK, unroll=4)
            def _(e):
                wv = plsc.load_gather(w_v[q], [jnp.full((LANES,), e, jnp.int32)])
                for g in range(D // (2 * LANES)):
                    xi = rg[p][e, pl.ds(g * LANES, LANES)]
                    lo = plsc.bitcast(xi << 16, jnp.float32)
                    hi = plsc.bitcast(xi & jnp.int32(-65536), jnp.float32)
                    rs[p][e, pl.ds(g * 2 * LANES, LANES)] = lo * wv
                    rs[p][e, pl.ds(g * 2 * LANES + LANES, LANES)] = hi * wv

        def issue_scatter(q, p):
            pltpu.async_copy(rs[p], acc_sh.at[dst_v[q]], ssem[p], add=True)

        issue_idx(0, 0)
        issue_idx(1, 1)
        wait_idx(0)
        wait_idx(1)
        issue_gather(0, 0)
        issue_gather(1, 1)

        def slot(i, q, p, qn):
            issue_idx(i + 2, qn)
            wait_gather(p)
            scale(q, p)
            issue_scatter(q, p)
            wait_idx(qn)
            issue_gather(qn, p)

        slot(0, 0, 0, 2)
        slot(1, 1, 1, 3)

        @pl.loop(0, (NCHUNK - 5) // 4)
        def _(j):
            i = 4 * j + 2
            for k in range(4):
                q = (2 + k) % 4
                p = k % 2
                wait_scatter(p)
                slot(i + k, q, p, k)

        wait_scatter(0)
        slot(NCHUNK - 3, 2, 0, 0)
        wait_scatter(1)
        wait_gather(1)
        scale(3, 1)
        issue_scatter(3, 1)
        wait_scatter(0)
        wait_gather(0)
        scale(0, 0)
        issue_scatter(0, 0)

        wait_scatter(1)
        wait_scatter(0)
        plsc.subcore_barrier()

        out_base = c * N_NODES + row_off

        @pl.when(s < NS - 1)
        def _():
            pltpu.sync_copy(acc_sh.at[pl.ds(row_off, ROWS_MAIN)],
                            out_hbm.at[pl.ds(out_base, ROWS_MAIN)])

        @pl.when(s == NS - 1)
        def _():
            pltpu.sync_copy(acc_sh.at[pl.ds(row_off, ROWS_LAST)],
                            out_hbm.at[pl.ds(out_base, ROWS_LAST)])

    return spmm(dst, src, wgt, features, zeros)


def _tc_combine_body(parts_ref, f_ref, w1_ref, w2_ref, bias_ref, out_ref):
    x = parts_ref[0] + parts_ref[1]
    f = f_ref[...]
    out_ref[...] = (
        jnp.dot(f + x, w1_ref[...], preferred_element_type=jnp.float32)
        + jnp.dot(x * f, w2_ref[...], preferred_element_type=jnp.float32)
        + bias_ref[...]
    )


def _tc_combine(parts, features, W1, W2, bias):
    blk = 1000
    grid = (N_NODES // blk,)
    return pl.pallas_call(
        _tc_combine_body,
        grid=grid,
        in_specs=[
            pl.BlockSpec((2, blk, D), lambda i: (0, i, 0)),
            pl.BlockSpec((blk, D), lambda i: (i, 0)),
            pl.BlockSpec((D, D), lambda i: (0, 0)),
            pl.BlockSpec((D, D), lambda i: (0, 0)),
            pl.BlockSpec((1, D), lambda i: (0, 0)),
        ],
        out_specs=pl.BlockSpec((blk, D), lambda i: (i, 0)),
        out_shape=jax.ShapeDtypeStruct((N_NODES, D), jnp.float32),
    )(parts, features, W1, W2, bias)


@jax.jit
def kernel(edge_index, edge_weight, features, W1, b1, W2, b2):
    dst = edge_index[0]
    src = edge_index[1]
    f_bf = features.astype(jnp.bfloat16)[:, _PERM]
    f_i32 = lax.bitcast_convert_type(
        f_bf.reshape(N_NODES, D // 2, 2), jnp.int32)
    zeros = jnp.zeros((N_NODES, D), jnp.float32)
    parts = _sc_spmm(dst, src, edge_weight, f_i32, zeros)
    parts = parts.reshape(NC, N_NODES, D)
    bias = (b1 + b2).reshape(1, D)
    return _tc_combine(parts, features, W1, W2, bias)


# speedup vs baseline: 11.3402x; 1.0301x over previous
import dataclasses
import functools

import jax
import jax.numpy as jnp
import numpy as np
from jax import lax
from jax.experimental import pallas as pl
from jax.experimental.pallas import tpu as pltpu
from jax.experimental.pallas import tpu_sc as plsc

N_NODES = 10000
N_EDGES = 320000
D = 128

_PERM = np.empty((D,), dtype=np.int32)
for _g in range(D // 32):
    for _k in range(16):
        _PERM[32 * _g + 2 * _k] = 32 * _g + _k
        _PERM[32 * _g + 2 * _k + 1] = 32 * _g + 16 + _k

NC = 2
NS = 16
NW = NC * NS
LANES = 16
CHUNK = 80
EDGES_PER_TILE = N_EDGES // NW
NCHUNK = EDGES_PER_TILE // CHUNK
ROWS_MAIN = 624
ROWS_LAST = N_NODES - (NS - 1) * ROWS_MAIN


def _sc_spmm(dst, src, wgt, features):
    mesh = plsc.VectorSubcoreMesh(
        core_axis_name="c", subcore_axis_name="s", num_cores=NC, num_subcores=NS
    )
    cp = pltpu.CompilerParams()
    if "needs_layout_passes" in pltpu.CompilerParams.__dataclass_fields__:
        cp = dataclasses.replace(cp, needs_layout_passes=False)
    if "use_tc_tiling_on_sc" in pltpu.CompilerParams.__dataclass_fields__:
        cp = dataclasses.replace(cp, use_tc_tiling_on_sc=False)

    @functools.partial(
        pl.kernel,
        compiler_params=cp,
        out_type=jax.ShapeDtypeStruct((NC * N_NODES, D), jnp.float32),
        mesh=mesh,
        scratch_types=[
            [pltpu.VMEM((CHUNK,), jnp.int32) for _ in range(4)],
            [pltpu.VMEM((CHUNK,), jnp.int32) for _ in range(4)],
            [pltpu.VMEM((CHUNK,), jnp.float32) for _ in range(4)],
            [pltpu.VMEM((CHUNK, D // 2), jnp.int32) for _ in range(2)],
            [pltpu.VMEM((CHUNK, D), jnp.float32) for _ in range(2)],
            pltpu.VMEM_SHARED((N_NODES, D), jnp.float32),
            [pltpu.SemaphoreType.DMA for _ in range(4)],
            [pltpu.SemaphoreType.DMA for _ in range(2)],
            [pltpu.SemaphoreType.DMA for _ in range(2)],
        ],
    )
    def spmm(dst_hbm, src_hbm, w_hbm, f_hbm, out_hbm,
             src_v, dst_v, w_v, rg, rs, acc_sh, isem, gsem, ssem):
        c = lax.axis_index("c")
        s = lax.axis_index("s")
        t = c * NS + s
        tile_base = t * EDGES_PER_TILE

        row_off = s * ROWS_MAIN

        @pl.loop(0, CHUNK)
        def _(e):
            for g in range(D // LANES):
                rs[0][e, pl.ds(g * LANES, LANES)] = jnp.zeros((LANES,),
                                                              jnp.float32)

        for k in range(7):
            pltpu.sync_copy(rs[0], acc_sh.at[pl.ds(row_off + k * CHUNK, CHUNK)])

        @pl.when(s < NS - 1)
        def _():
            pltpu.sync_copy(rs[0].at[pl.ds(0, ROWS_MAIN - 7 * CHUNK)],
                            acc_sh.at[pl.ds(row_off + 7 * CHUNK,
                                            ROWS_MAIN - 7 * CHUNK)])

        @pl.when(s == NS - 1)
        def _():
            pltpu.sync_copy(rs[0], acc_sh.at[pl.ds(row_off + 7 * CHUNK, CHUNK)])

        plsc.subcore_barrier()

        def issue_idx(i, q):
            base = tile_base + i * CHUNK
            pltpu.async_copy(src_hbm.at[pl.ds(base, CHUNK)], src_v[q], isem[q])
            pltpu.async_copy(dst_hbm.at[pl.ds(base, CHUNK)], dst_v[q], isem[q])
            pltpu.async_copy(w_hbm.at[pl.ds(base, CHUNK)], w_v[q], isem[q])

        def wait_idx(q):
            pltpu.make_async_copy(src_hbm.at[pl.ds(0, CHUNK)], src_v[q],
                                  isem[q]).wait()
            pltpu.make_async_copy(dst_hbm.at[pl.ds(0, CHUNK)], dst_v[q],
                                  isem[q]).wait()
            pltpu.make_async_copy(w_hbm.at[pl.ds(0, CHUNK)], w_v[q],
                                  isem[q]).wait()

        def issue_gather(q, p):
            pltpu.async_copy(f_hbm.at[src_v[q]], rg[p], gsem[p])

        def wait_gather(p):
            pltpu.make_async_copy(f_hbm.at[src_v[0]], rg[p], gsem[p]).wait()

        def wait_scatter(p):
            pltpu.make_async_copy(rs[p], acc_sh.at[dst_v[0]], ssem[p]).wait()

        def scale(q, p):
            @plsc.parallel_loop(0, CHUNK, unroll=8)
            def _(e):
                wv = plsc.load_gather(w_v[q], [jnp.full((LANES,), e, jnp.int32)])
                for g in range(D // (2 * LANES)):
                    xi = rg[p][e, pl.ds(g * LANES, LANES)]
                    lo = plsc.bitcast(xi << 16, jnp.float32)
                    hi = plsc.bitcast(xi & jnp.int32(-65536), jnp.float32)
                    rs[p][e, pl.ds(g * 2 * LANES, LANES)] = lo * wv
                    rs[p][e, pl.ds(g * 2 * LANES + LANES, LANES)] = hi * wv

        def issue_scatter(q, p):
            pltpu.async_copy(rs[p], acc_sh.at[dst_v[q]], ssem[p], add=True)

        issue_idx(0, 0)
        issue_idx(1, 1)
        wait_idx(0)
        wait_idx(1)
        issue_gather(0, 0)
        issue_gather(1, 1)

        def slot(i, q, p, qn):
            issue_idx(i + 2, qn)
            wait_gather(p)
            scale(q, p)
            issue_scatter(q, p)
            wait_idx(qn)
            issue_gather(qn, p)

        slot(0, 0, 0, 2)
        slot(1, 1, 1, 3)

        @pl.loop(0, (NCHUNK - 5) // 4)
        def _(j):
            i = 4 * j + 2
            for k in range(4):
                q = (2 + k) % 4
                p = k % 2
                wait_scatter(p)
                slot(i + k, q, p, k)

        wait_scatter(0)
        slot(NCHUNK - 3, 2, 0, 0)
        wait_scatter(1)
        wait_gather(1)
        scale(3, 1)
        issue_scatter(3, 1)
        wait_scatter(0)
        wait_gather(0)
        scale(0, 0)
        issue_scatter(0, 0)

        wait_scatter(1)
        wait_scatter(0)
        plsc.subcore_barrier()

        out_base = c * N_NODES + row_off

        @pl.when(s < NS - 1)
        def _():
            pltpu.sync_copy(acc_sh.at[pl.ds(row_off, ROWS_MAIN)],
                            out_hbm.at[pl.ds(out_base, ROWS_MAIN)])

        @pl.when(s == NS - 1)
        def _():
            pltpu.sync_copy(acc_sh.at[pl.ds(row_off, ROWS_LAST)],
                            out_hbm.at[pl.ds(out_base, ROWS_LAST)])

    return spmm(dst, src, wgt, features)


def _tc_combine_body(parts_ref, f_ref, w1_ref, w2_ref, bias_ref, out_ref):
    x = parts_ref[0] + parts_ref[1]
    f = f_ref[...]
    out_ref[...] = (
        jnp.dot(f + x, w1_ref[...], preferred_element_type=jnp.float32)
        + jnp.dot(x * f, w2_ref[...], preferred_element_type=jnp.float32)
        + bias_ref[...]
    )


def _tc_combine(parts, features, W1, W2, bias):
    blk = 1000
    grid = (N_NODES // blk,)
    return pl.pallas_call(
        _tc_combine_body,
        grid=grid,
        in_specs=[
            pl.BlockSpec((2, blk, D), lambda i: (0, i, 0)),
            pl.BlockSpec((blk, D), lambda i: (i, 0)),
            pl.BlockSpec((D, D), lambda i: (0, 0)),
            pl.BlockSpec((D, D), lambda i: (0, 0)),
            pl.BlockSpec((1, D), lambda i: (0, 0)),
        ],
        out_specs=pl.BlockSpec((blk, D), lambda i: (i, 0)),
        out_shape=jax.ShapeDtypeStruct((N_NODES, D), jnp.float32),
    )(parts, features, W1, W2, bias)


@jax.jit
def kernel(edge_index, edge_weight, features, W1, b1, W2, b2):
    dst = edge_index[0]
    src = edge_index[1]
    f_bf = features.astype(jnp.bfloat16)[:, _PERM]
    f_i32 = lax.bitcast_convert_type(
        f_bf.reshape(N_NODES, D // 2, 2), jnp.int32)
    parts = _sc_spmm(dst, src, edge_weight, f_i32)
    parts = parts.reshape(NC, N_NODES, D)
    bias = (b1 + b2).reshape(1, D)
    return _tc_combine(parts, features, W1, W2, bias)


# speedup vs baseline: 14.0748x; 1.2411x over previous
import dataclasses
import functools

import jax
import jax.numpy as jnp
import numpy as np
from jax import lax
from jax.experimental import pallas as pl
from jax.experimental.pallas import tpu as pltpu
from jax.experimental.pallas import tpu_sc as plsc

N_NODES = 10000
N_EDGES = 320000
D = 128

_TPOS = np.empty((D,), dtype=np.int32)
for _g in range(4):
    for _k in range(16):
        _TPOS[16 * _g + _k] = 32 * _g + _k
        _TPOS[64 + 16 * _g + _k] = 32 * _g + 16 + _k
_INV = np.argsort(_TPOS).astype(np.int32)

NC = 2
NS = 16
NW = NC * NS
LANES = 16
CHUNK = 80
EDGES_PER_TILE = N_EDGES // NW
NCHUNK = EDGES_PER_TILE // CHUNK
ROWS_MAIN = 624
ROWS_LAST = N_NODES - (NS - 1) * ROWS_MAIN


def _sc_spmm(ep, wgt, features):
    mesh = plsc.VectorSubcoreMesh(
        core_axis_name="c", subcore_axis_name="s", num_cores=NC, num_subcores=NS
    )
    cp = pltpu.CompilerParams()
    if "needs_layout_passes" in pltpu.CompilerParams.__dataclass_fields__:
        cp = dataclasses.replace(cp, needs_layout_passes=False)
    if "use_tc_tiling_on_sc" in pltpu.CompilerParams.__dataclass_fields__:
        cp = dataclasses.replace(cp, use_tc_tiling_on_sc=False)

    @functools.partial(
        pl.kernel,
        compiler_params=cp,
        out_type=jax.ShapeDtypeStruct((NC * N_NODES, D), jnp.float32),
        mesh=mesh,
        scratch_types=[
            [pltpu.VMEM((CHUNK,), jnp.int32) for _ in range(4)],
            [pltpu.VMEM((CHUNK,), jnp.int32) for _ in range(4)],
            [pltpu.VMEM((CHUNK,), jnp.int32) for _ in range(4)],
            [pltpu.VMEM((CHUNK,), jnp.int32) for _ in range(4)],
            [pltpu.VMEM((CHUNK, D // 2), jnp.int32) for _ in range(2)],
            [pltpu.VMEM((CHUNK, D), jnp.float32) for _ in range(2)],
            pltpu.VMEM_SHARED((N_NODES, D), jnp.float32),
            [pltpu.SemaphoreType.DMA for _ in range(4)],
            [pltpu.SemaphoreType.DMA for _ in range(2)],
            [pltpu.SemaphoreType.DMA for _ in range(2)],
        ],
    )
    def spmm(ep_hbm, w_hbm, f_hbm, out_hbm,
             src_v, dst_v, pk_v, w_v, rg, rs, acc_sh, isem, gsem, ssem):
        c = lax.axis_index("c")
        s = lax.axis_index("s")
        t = c * NS + s
        tile_base = t * EDGES_PER_TILE

        row_off = s * ROWS_MAIN

        @pl.loop(0, CHUNK)
        def _(e):
            for g in range(D // LANES):
                rs[0][e, pl.ds(g * LANES, LANES)] = jnp.zeros((LANES,),
                                                              jnp.float32)

        for k in range(7):
            pltpu.sync_copy(rs[0], acc_sh.at[pl.ds(row_off + k * CHUNK, CHUNK)])

        @pl.when(s < NS - 1)
        def _():
            pltpu.sync_copy(rs[0].at[pl.ds(0, ROWS_MAIN - 7 * CHUNK)],
                            acc_sh.at[pl.ds(row_off + 7 * CHUNK,
                                            ROWS_MAIN - 7 * CHUNK)])

        @pl.when(s == NS - 1)
        def _():
            pltpu.sync_copy(rs[0], acc_sh.at[pl.ds(row_off + 7 * CHUNK, CHUNK)])

        plsc.subcore_barrier()

        def issue_idx(i, q):
            base = tile_base + i * CHUNK
            pltpu.async_copy(ep_hbm.at[pl.ds(base, CHUNK)], pk_v[q], isem[q])
            pltpu.async_copy(w_hbm.at[pl.ds(base, CHUNK)], w_v[q], isem[q])

        def wait_idx(q):
            pltpu.make_async_copy(ep_hbm.at[pl.ds(0, CHUNK)], pk_v[q],
                                  isem[q]).wait()
            pltpu.make_async_copy(w_hbm.at[pl.ds(0, CHUNK)], w_v[q],
                                  isem[q]).wait()
            for u in range(CHUNK // LANES):
                pk = pk_v[q][pl.ds(u * LANES, LANES)]
                src_v[q][pl.ds(u * LANES, LANES)] = pk & jnp.int32(0xFFFF)
                dst_v[q][pl.ds(u * LANES, LANES)] = lax.shift_right_logical(
                    pk, 16)

        def issue_gather(q, p):
            pltpu.async_copy(f_hbm.at[src_v[q]], rg[p], gsem[p])

        def wait_gather(p):
            pltpu.make_async_copy(f_hbm.at[src_v[0]], rg[p], gsem[p]).wait()

        def wait_scatter(p):
            pltpu.make_async_copy(rs[p], acc_sh.at[dst_v[0]], ssem[p]).wait()

        def scale(q, p):
            @plsc.parallel_loop(0, CHUNK, unroll=16)
            def _(e):
                wv = plsc.load_gather(w_v[q], [jnp.full((LANES,), e, jnp.int32)])
                wb = plsc.bitcast(wv, jnp.bfloat16)
                for g in range(D // (2 * LANES)):
                    xi = rg[p][e, pl.ds(g * LANES, LANES)]
                    pi = plsc.bitcast(plsc.bitcast(xi, jnp.bfloat16) * wb,
                                      jnp.int32)
                    lo = plsc.bitcast(pi << 16, jnp.float32)
                    hi = plsc.bitcast(pi, jnp.float32)
                    rs[p][e, pl.ds(g * 2 * LANES, LANES)] = lo
                    rs[p][e, pl.ds(g * 2 * LANES + LANES, LANES)] = hi

        def issue_scatter(q, p):
            pltpu.async_copy(rs[p], acc_sh.at[dst_v[q]], ssem[p], add=True)

        issue_idx(0, 0)
        issue_idx(1, 1)
        wait_idx(0)
        wait_idx(1)
        issue_gather(0, 0)
        issue_gather(1, 1)

        def slot(i, q, p, qn):
            issue_idx(i + 2, qn)
            wait_gather(p)
            scale(q, p)
            issue_scatter(q, p)
            wait_idx(qn)
            issue_gather(qn, p)

        slot(0, 0, 0, 2)
        slot(1, 1, 1, 3)

        @pl.loop(0, (NCHUNK - 5) // 4)
        def _(j):
            i = 4 * j + 2
            for k in range(4):
                q = (2 + k) % 4
                p = k % 2
                wait_scatter(p)
                slot(i + k, q, p, k)

        wait_scatter(0)
        slot(NCHUNK - 3, 2, 0, 0)
        wait_scatter(1)
        wait_gather(1)
        scale(3, 1)
        issue_scatter(3, 1)
        wait_scatter(0)
        wait_gather(0)
        scale(0, 0)
        issue_scatter(0, 0)

        wait_scatter(1)
        wait_scatter(0)
        plsc.subcore_barrier()

        out_base = c * N_NODES + row_off

        @pl.when(s < NS - 1)
        def _():
            pltpu.sync_copy(acc_sh.at[pl.ds(row_off, ROWS_MAIN)],
                            out_hbm.at[pl.ds(out_base, ROWS_MAIN)])

        @pl.when(s == NS - 1)
        def _():
            pltpu.sync_copy(acc_sh.at[pl.ds(row_off, ROWS_LAST)],
                            out_hbm.at[pl.ds(out_base, ROWS_LAST)])

    return spmm(ep, wgt, features)


def _tc_combine_body(parts_ref, f_ref, w1_ref, w2_ref, bias_ref, out_ref):
    x = parts_ref[0] + parts_ref[1]
    f = f_ref[...]
    out_ref[...] = (
        jnp.dot(f + x, w1_ref[...], preferred_element_type=jnp.float32)
        + jnp.dot(x * f, w2_ref[...], preferred_element_type=jnp.float32)
        + bias_ref[...]
    )


def _tc_combine(parts, features, W1, W2, bias):
    blk = 2000
    grid = (N_NODES // blk,)
    return pl.pallas_call(
        _tc_combine_body,
        grid=grid,
        in_specs=[
            pl.BlockSpec((2, blk, D), lambda i: (0, i, 0)),
            pl.BlockSpec((blk, D), lambda i: (i, 0)),
            pl.BlockSpec((D, D), lambda i: (0, 0)),
            pl.BlockSpec((D, D), lambda i: (0, 0)),
            pl.BlockSpec((1, D), lambda i: (0, 0)),
        ],
        out_specs=pl.BlockSpec((blk, D), lambda i: (i, 0)),
        out_shape=jax.ShapeDtypeStruct((N_NODES, D), jnp.float32),
    )(parts, features, W1, W2, bias)


def _tc_pack_body(f_ref, out_ref):
    xi = lax.bitcast_convert_type(f_ref[...], jnp.int32)
    a = xi[:, : D // 2] + jnp.int32(0x8000)
    b = xi[:, D // 2:] + jnp.int32(0x8000)
    out_ref[...] = (lax.shift_right_logical(a, 16)
                    | (b & jnp.int32(-65536)))


def _tc_pack(features):
    blk = 2000
    return pl.pallas_call(
        _tc_pack_body,
        grid=(N_NODES // blk,),
        in_specs=[pl.BlockSpec((blk, D), lambda i: (i, 0))],
        out_specs=pl.BlockSpec((blk, D // 2), lambda i: (i, 0)),
        out_shape=jax.ShapeDtypeStruct((N_NODES, D // 2), jnp.int32),
    )(features)


def _tc_edge_pack_body(ei_ref, w_ref, ep_ref, wp_ref):
    ep_ref[...] = (ei_ref[0] << 16) | ei_ref[1]
    wb = lax.shift_right_logical(
        lax.bitcast_convert_type(w_ref[...], jnp.int32) + jnp.int32(0x8000),
        16)
    wp_ref[...] = (wb << 16) | wb


def _tc_edge_pack(edge_index, edge_weight):
    return pl.pallas_call(
        _tc_edge_pack_body,
        out_shape=[jax.ShapeDtypeStruct((N_EDGES,), jnp.int32),
                   jax.ShapeDtypeStruct((N_EDGES,), jnp.int32)],
    )(edge_index, edge_weight)


@jax.jit
def kernel(edge_index, edge_weight, features, W1, b1, W2, b2):
    f_i32 = _tc_pack(features)
    ep, wpk = _tc_edge_pack(edge_index, edge_weight)
    parts = _sc_spmm(ep, wpk, f_i32)
    parts = parts.reshape(NC, N_NODES, D)
    f_t = features[:, _INV]
    w1_t = W1[_INV, :]
    w2_t = W2[_INV, :]
    bias = (b1 + b2).reshape(1, D)
    return _tc_combine(parts, f_t, w1_t, w2_t, bias)


# speedup vs baseline: 14.4630x; 1.0276x over previous
import dataclasses
import functools

import jax
import jax.numpy as jnp
import numpy as np
from jax import lax
from jax.experimental import pallas as pl
from jax.experimental.pallas import tpu as pltpu
from jax.experimental.pallas import tpu_sc as plsc

N_NODES = 10000
N_EDGES = 320000
D = 128

_TPOS = np.empty((D,), dtype=np.int32)
for _g in range(4):
    for _k in range(16):
        _TPOS[16 * _g + _k] = 32 * _g + _k
        _TPOS[64 + 16 * _g + _k] = 32 * _g + 16 + _k
_INV = np.argsort(_TPOS).astype(np.int32)

NC = 2
NS = 16
NW = NC * NS
LANES = 16
CHUNK = 80
EDGES_PER_TILE = N_EDGES // NW
NCHUNK = EDGES_PER_TILE // CHUNK
ROWS_MAIN = 624
ROWS_LAST = N_NODES - (NS - 1) * ROWS_MAIN


def _sc_spmm(ep, wgt, features):
    mesh = plsc.VectorSubcoreMesh(
        core_axis_name="c", subcore_axis_name="s", num_cores=NC, num_subcores=NS
    )
    cp = pltpu.CompilerParams()
    if "needs_layout_passes" in pltpu.CompilerParams.__dataclass_fields__:
        cp = dataclasses.replace(cp, needs_layout_passes=False)
    if "use_tc_tiling_on_sc" in pltpu.CompilerParams.__dataclass_fields__:
        cp = dataclasses.replace(cp, use_tc_tiling_on_sc=False)

    @functools.partial(
        pl.kernel,
        compiler_params=cp,
        out_type=jax.ShapeDtypeStruct((NC * N_NODES, D), jnp.float32),
        mesh=mesh,
        scratch_types=[
            [pltpu.VMEM((CHUNK,), jnp.int32) for _ in range(4)],
            [pltpu.VMEM((CHUNK,), jnp.int32) for _ in range(4)],
            [pltpu.VMEM((CHUNK,), jnp.int32) for _ in range(4)],
            [pltpu.VMEM((CHUNK,), jnp.float32) for _ in range(4)],
            [pltpu.VMEM((CHUNK, D // 2), jnp.int32) for _ in range(2)],
            [pltpu.VMEM((CHUNK, D), jnp.float32) for _ in range(2)],
            pltpu.VMEM_SHARED((N_NODES, D), jnp.float32),
            [pltpu.SemaphoreType.DMA for _ in range(4)],
            [pltpu.SemaphoreType.DMA for _ in range(2)],
            [pltpu.SemaphoreType.DMA for _ in range(2)],
        ],
    )
    def spmm(ep_hbm, w_hbm, f_hbm, out_hbm,
             src_v, dst_v, pk_v, w_v, rg, rs, acc_sh, isem, gsem, ssem):
        c = lax.axis_index("c")
        s = lax.axis_index("s")
        t = c * NS + s
        tile_base = t * EDGES_PER_TILE

        row_off = s * ROWS_MAIN

        @pl.loop(0, CHUNK)
        def _(e):
            for g in range(D // LANES):
                rs[0][e, pl.ds(g * LANES, LANES)] = jnp.zeros((LANES,),
                                                              jnp.float32)

        for k in range(7):
            pltpu.sync_copy(rs[0], acc_sh.at[pl.ds(row_off + k * CHUNK, CHUNK)])

        @pl.when(s < NS - 1)
        def _():
            pltpu.sync_copy(rs[0].at[pl.ds(0, ROWS_MAIN - 7 * CHUNK)],
                            acc_sh.at[pl.ds(row_off + 7 * CHUNK,
                                            ROWS_MAIN - 7 * CHUNK)])

        @pl.when(s == NS - 1)
        def _():
            pltpu.sync_copy(rs[0], acc_sh.at[pl.ds(row_off + 7 * CHUNK, CHUNK)])

        plsc.subcore_barrier()

        def issue_idx(i, q):
            base = tile_base + i * CHUNK
            pltpu.async_copy(ep_hbm.at[pl.ds(base, CHUNK)], pk_v[q], isem[q])
            pltpu.async_copy(w_hbm.at[pl.ds(base, CHUNK)], w_v[q], isem[q])

        def wait_idx(q):
            pltpu.make_async_copy(ep_hbm.at[pl.ds(0, CHUNK)], pk_v[q],
                                  isem[q]).wait()
            pltpu.make_async_copy(w_hbm.at[pl.ds(0, CHUNK)], w_v[q],
                                  isem[q]).wait()
            for u in range(CHUNK // LANES):
                pk = pk_v[q][pl.ds(u * LANES, LANES)]
                src_v[q][pl.ds(u * LANES, LANES)] = pk & jnp.int32(0xFFFF)
                dst_v[q][pl.ds(u * LANES, LANES)] = lax.shift_right_logical(
                    pk, 16)

        def issue_gather(q, p):
            pltpu.async_copy(f_hbm.at[src_v[q]], rg[p], gsem[p])

        def wait_gather(p):
            pltpu.make_async_copy(f_hbm.at[src_v[0]], rg[p], gsem[p]).wait()

        def wait_scatter(p):
            pltpu.make_async_copy(rs[p], acc_sh.at[dst_v[0]], ssem[p]).wait()

        def scale(q, p):
            @plsc.parallel_loop(0, CHUNK, unroll=16)
            def _(e):
                wv = plsc.load_gather(w_v[q], [jnp.full((LANES,), e, jnp.int32)])
                for g in range(D // (2 * LANES)):
                    xi = rg[p][e, pl.ds(g * LANES, LANES)]
                    lo = plsc.bitcast(xi << 16, jnp.float32)
                    hi = plsc.bitcast(xi, jnp.float32)
                    rs[p][e, pl.ds(g * 2 * LANES, LANES)] = lo * wv
                    rs[p][e, pl.ds(g * 2 * LANES + LANES, LANES)] = hi * wv

        def issue_scatter(q, p):
            pltpu.async_copy(rs[p], acc_sh.at[dst_v[q]], ssem[p], add=True)

        issue_idx(0, 0)
        issue_idx(1, 1)
        wait_idx(0)
        wait_idx(1)
        issue_gather(0, 0)
        issue_gather(1, 1)

        def slot(i, q, p, qn):
            issue_idx(i + 2, qn)
            wait_gather(p)
            scale(q, p)
            issue_scatter(q, p)
            wait_idx(qn)
            issue_gather(qn, p)

        slot(0, 0, 0, 2)
        slot(1, 1, 1, 3)

        @pl.loop(0, (NCHUNK - 5) // 4)
        def _(j):
            i = 4 * j + 2
            for k in range(4):
                q = (2 + k) % 4
                p = k % 2
                wait_scatter(p)
                slot(i + k, q, p, k)

        wait_scatter(0)
        slot(NCHUNK - 3, 2, 0, 0)
        wait_scatter(1)
        wait_gather(1)
        scale(3, 1)
        issue_scatter(3, 1)
        wait_scatter(0)
        wait_gather(0)
        scale(0, 0)
        issue_scatter(0, 0)

        wait_scatter(1)
        wait_scatter(0)
        plsc.subcore_barrier()

        out_base = c * N_NODES + row_off

        @pl.when(s < NS - 1)
        def _():
            pltpu.sync_copy(acc_sh.at[pl.ds(row_off, ROWS_MAIN)],
                            out_hbm.at[pl.ds(out_base, ROWS_MAIN)])

        @pl.when(s == NS - 1)
        def _():
            pltpu.sync_copy(acc_sh.at[pl.ds(row_off, ROWS_LAST)],
                            out_hbm.at[pl.ds(out_base, ROWS_LAST)])

    return spmm(ep, wgt, features)


def _tc_combine_body(parts_ref, f_ref, w1_ref, w2_ref, bias_ref, out_ref):
    x = parts_ref[0] + parts_ref[1]
    f = f_ref[...]
    out_ref[...] = (
        jnp.dot(f + x, w1_ref[...], preferred_element_type=jnp.float32)
        + jnp.dot(x * f, w2_ref[...], preferred_element_type=jnp.float32)
        + bias_ref[...]
    )


def _tc_combine(parts, features, W1, W2, bias):
    blk = 2000
    grid = (N_NODES // blk,)
    return pl.pallas_call(
        _tc_combine_body,
        grid=grid,
        in_specs=[
            pl.BlockSpec((2, blk, D), lambda i: (0, i, 0)),
            pl.BlockSpec((blk, D), lambda i: (i, 0)),
            pl.BlockSpec((D, D), lambda i: (0, 0)),
            pl.BlockSpec((D, D), lambda i: (0, 0)),
            pl.BlockSpec((1, D), lambda i: (0, 0)),
        ],
        out_specs=pl.BlockSpec((blk, D), lambda i: (i, 0)),
        out_shape=jax.ShapeDtypeStruct((N_NODES, D), jnp.float32),
    )(parts, features, W1, W2, bias)


def _tc_pack_body(f_ref, out_ref):
    xi = lax.bitcast_convert_type(f_ref[...], jnp.int32)
    a = xi[:, : D // 2] + jnp.int32(0x8000)
    b = xi[:, D // 2:] + jnp.int32(0x8000)
    out_ref[...] = (lax.shift_right_logical(a, 16)
                    | (b & jnp.int32(-65536)))


def _tc_pack(features):
    blk = 2000
    return pl.pallas_call(
        _tc_pack_body,
        grid=(N_NODES // blk,),
        in_specs=[pl.BlockSpec((blk, D), lambda i: (i, 0))],
        out_specs=pl.BlockSpec((blk, D // 2), lambda i: (i, 0)),
        out_shape=jax.ShapeDtypeStruct((N_NODES, D // 2), jnp.int32),
    )(features)


def _tc_edge_pack_body(ei_ref, ep_ref):
    ep_ref[...] = (ei_ref[0] << 16) | ei_ref[1]


def _tc_edge_pack(edge_index):
    return pl.pallas_call(
        _tc_edge_pack_body,
        out_shape=jax.ShapeDtypeStruct((N_EDGES,), jnp.int32),
    )(edge_index)


@jax.jit
def kernel(edge_index, edge_weight, features, W1, b1, W2, b2):
    f_i32 = _tc_pack(features)
    ep = _tc_edge_pack(edge_index)
    parts = _sc_spmm(ep, edge_weight, f_i32)
    parts = parts.reshape(NC, N_NODES, D)
    f_t = features[:, _INV]
    w1_t = W1[_INV, :]
    w2_t = W2[_INV, :]
    bias = (b1 + b2).reshape(1, D)
    return _tc_combine(parts, f_t, w1_t, w2_t, bias)
